# Initial kernel scaffold; baseline (speedup 1.0000x reference)
import jax
import jax.numpy as jnp
from jax.experimental import pallas as pl


def kernel(bi_loc_data, bi_conf_data, multi_loc_data, multi_conf_data, prior_data):
    raise NotImplementedError("write your pallas kernel here")


# speedup vs baseline: 9.4439x; 9.4439x over previous
import jax
import jax.numpy as jnp
from jax.experimental import pallas as pl

_NUM_CLASSES = 21
_TOP_K = 200
_CONF_THRESH = 0.5
_NMS_THRESH = 0.45
_N = 20000
_NP = 20480
_R = _NP // 128
_C = 128
_S = 256


def _detect_body(bl_ref, bc_ref, ml_ref, cf_ref, pr_ref, out_ref):
    cl = pl.program_id(1)
    zero_slab = jnp.zeros((1, 1, 8, _S), jnp.float32)
    out_ref[...] = zero_slab

    @pl.when(cl > 0)
    def _work():
        flag = bc_ref[0, 1] >= bc_ref[0, 0]

        pcx, pcy, pw, ph = pr_ref[0], pr_ref[1], pr_ref[2], pr_ref[3]
        blx, bly, blw, blh = bl_ref[0, 0], bl_ref[0, 1], bl_ref[0, 2], bl_ref[0, 3]
        mlx, mly, mlw, mlh = ml_ref[0, 0], ml_ref[0, 1], ml_ref[0, 2], ml_ref[0, 3]

        bcx = pcx + blx * 0.1 * pw
        bcy = pcy + bly * 0.1 * ph
        bw = pw * jnp.exp(blw * 0.2)
        bh = ph * jnp.exp(blh * 0.2)
        ax1 = bcx - bw / 2.0
        ay1 = bcy - bh / 2.0
        ax2 = ax1 + bw
        ay2 = ay1 + bh
        rcx = (ax1 + ax2) / 2.0
        rcy = (ay1 + ay2) / 2.0
        rw = ax2 - ax1
        rh = ay2 - ay1

        ocx = rcx + mlx * 0.1 * rw
        ocy = rcy + mly * 0.1 * rh
        ow = rw * jnp.exp(mlw * 0.2)
        oh = rh * jnp.exp(mlh * 0.2)
        bx1 = ocx - ow / 2.0
        by1 = ocy - oh / 2.0
        bx2 = bx1 + ow
        by2 = by1 + oh

        conf = cf_ref[0, 0]
        neg_inf = jnp.float32(-jnp.inf)
        masked = jnp.where(flag & (conf > _CONF_THRESH), conf, neg_inf)

        iota = (jax.lax.broadcasted_iota(jnp.int32, (_R, _C), 0) * _C
                + jax.lax.broadcasted_iota(jnp.int32, (_R, _C), 1))
        slot = jax.lax.broadcasted_iota(jnp.int32, (1, _S), 1)

        def sel_body(t, carry):
            sc, ts, tx1, ty1, tx2, ty2 = carry
            val = jnp.max(sc)
            idx = jnp.max(jnp.where(sc == val, iota, -1))
            m = iota == idx
            gx1 = jnp.sum(jnp.where(m, bx1, 0.0))
            gy1 = jnp.sum(jnp.where(m, by1, 0.0))
            gx2 = jnp.sum(jnp.where(m, bx2, 0.0))
            gy2 = jnp.sum(jnp.where(m, by2, 0.0))
            sel = slot == t
            ts = jnp.where(sel, val, ts)
            tx1 = jnp.where(sel, gx1, tx1)
            ty1 = jnp.where(sel, gy1, ty1)
            tx2 = jnp.where(sel, gx2, tx2)
            ty2 = jnp.where(sel, gy2, ty2)
            sc = jnp.where(m, neg_inf, sc)
            return sc, ts, tx1, ty1, tx2, ty2

        init = (masked,
                jnp.full((1, _S), neg_inf, jnp.float32),
                jnp.zeros((1, _S), jnp.float32),
                jnp.zeros((1, _S), jnp.float32),
                jnp.zeros((1, _S), jnp.float32),
                jnp.zeros((1, _S), jnp.float32))
        _, ts, tx1, ty1, tx2, ty2 = jax.lax.fori_loop(0, _TOP_K, sel_body, init)

        areav = (tx2 - tx1) * (ty2 - ty1)
        zcol = jnp.zeros((1, _S), jnp.float32)

        def nms_body(t, carry):
            supp, count, os, ox1, oy1, ox2, oy2 = carry
            selt = slot == t
            st = jnp.sum(jnp.where(selt, ts, 0.0))
            x1t = jnp.sum(jnp.where(selt, tx1, 0.0))
            y1t = jnp.sum(jnp.where(selt, ty1, 0.0))
            x2t = jnp.sum(jnp.where(selt, tx2, 0.0))
            y2t = jnp.sum(jnp.where(selt, ty2, 0.0))
            suppt = jnp.sum(jnp.where(selt, supp, 0.0)) > 0.0
            active = (st > _CONF_THRESH) & jnp.logical_not(suppt)

            xx1 = jnp.maximum(x1t, tx1)
            yy1 = jnp.maximum(y1t, ty1)
            xx2 = jnp.minimum(x2t, tx2)
            yy2 = jnp.minimum(y2t, ty2)
            w = jnp.clip(xx2 - xx1, 0.0, None)
            h = jnp.clip(yy2 - yy1, 0.0, None)
            inter = w * h
            areat = (x2t - x1t) * (y2t - y1t)
            iou = inter / (areat + areav - inter + 1e-12)
            supp = jnp.where(active & (iou > _NMS_THRESH) & (slot > t),
                             1.0, supp)

            put = active & (slot == count)
            os = jnp.where(put, st, os)
            ox1 = jnp.where(put, x1t, ox1)
            oy1 = jnp.where(put, y1t, oy1)
            ox2 = jnp.where(put, x2t, ox2)
            oy2 = jnp.where(put, y2t, oy2)
            count = count + active.astype(jnp.int32)
            return supp, count, os, ox1, oy1, ox2, oy2

        supp0 = jnp.zeros((1, _S), jnp.float32)
        _, _, os, ox1, oy1, ox2, oy2 = jax.lax.fori_loop(
            0, _TOP_K, nms_body,
            (supp0, jnp.int32(0), zcol, zcol, zcol, zcol, zcol))

        slab = jnp.concatenate([os, ox1, oy1, ox2, oy2, zcol, zcol, zcol],
                               axis=0)
        out_ref[...] = slab.reshape(1, 1, 8, _S)


def _planes(x, npad):
    b, n, k = x.shape
    xp = jnp.pad(x, ((0, 0), (0, npad - n), (0, 0)))
    return xp.transpose(0, 2, 1).reshape(b, k, _R, _C)


@jax.jit
def kernel(bi_loc_data, bi_conf_data, multi_loc_data, multi_conf_data, prior_data):
    num = bi_loc_data.shape[0]
    bl = _planes(bi_loc_data, _NP)
    ml = _planes(multi_loc_data, _NP)
    bc = _planes(bi_conf_data.reshape(num, _N, 2), _NP)
    cf = _planes(multi_conf_data.reshape(num, _N, _NUM_CLASSES), _NP)
    pr = jnp.pad(prior_data, ((0, _NP - _N), (0, 0))).T.reshape(4, _R, _C)

    slab = pl.pallas_call(
        _detect_body,
        grid=(num, _NUM_CLASSES),
        in_specs=[
            pl.BlockSpec((1, 4, _R, _C), lambda i, c: (i, 0, 0, 0)),
            pl.BlockSpec((1, 2, _R, _C), lambda i, c: (i, 0, 0, 0)),
            pl.BlockSpec((1, 4, _R, _C), lambda i, c: (i, 0, 0, 0)),
            pl.BlockSpec((1, 1, _R, _C), lambda i, c: (i, c, 0, 0)),
            pl.BlockSpec((4, _R, _C), lambda i, c: (0, 0, 0)),
        ],
        out_specs=pl.BlockSpec((1, 1, 8, _S), lambda i, c: (i, c, 0, 0)),
        out_shape=jax.ShapeDtypeStruct((num, _NUM_CLASSES, 8, _S),
                                       jnp.float32),
    )(bl, bc, ml, cf, pr)
    return slab[:, :, :5, :_TOP_K].transpose(0, 1, 3, 2)


# speedup vs baseline: 9.4465x; 1.0003x over previous
import jax
import jax.numpy as jnp
from jax.experimental import pallas as pl
from jax.experimental.pallas import tpu as pltpu

_NUM_CLASSES = 21
_TOP_K = 200
_CONF_THRESH = 0.5
_NMS_THRESH = 0.45
_N = 20000
_NP = 20480
_R = _NP // 128
_C = 128
_S = 256


def _detect_body(bl_ref, bc_ref, ml_ref, cf_ref, pr_ref, out_ref):
    cl = pl.program_id(1)
    zero_slab = jnp.zeros((1, 1, 8, _S), jnp.float32)
    out_ref[...] = zero_slab

    @pl.when(cl > 0)
    def _work():
        flag = bc_ref[0, 1] >= bc_ref[0, 0]

        pcx, pcy, pw, ph = pr_ref[0], pr_ref[1], pr_ref[2], pr_ref[3]
        blx, bly, blw, blh = bl_ref[0, 0], bl_ref[0, 1], bl_ref[0, 2], bl_ref[0, 3]
        mlx, mly, mlw, mlh = ml_ref[0, 0], ml_ref[0, 1], ml_ref[0, 2], ml_ref[0, 3]

        bcx = pcx + blx * 0.1 * pw
        bcy = pcy + bly * 0.1 * ph
        bw = pw * jnp.exp(blw * 0.2)
        bh = ph * jnp.exp(blh * 0.2)
        ax1 = bcx - bw / 2.0
        ay1 = bcy - bh / 2.0
        ax2 = ax1 + bw
        ay2 = ay1 + bh
        rcx = (ax1 + ax2) / 2.0
        rcy = (ay1 + ay2) / 2.0
        rw = ax2 - ax1
        rh = ay2 - ay1

        ocx = rcx + mlx * 0.1 * rw
        ocy = rcy + mly * 0.1 * rh
        ow = rw * jnp.exp(mlw * 0.2)
        oh = rh * jnp.exp(mlh * 0.2)
        bx1 = ocx - ow / 2.0
        by1 = ocy - oh / 2.0
        bx2 = bx1 + ow
        by2 = by1 + oh

        conf = cf_ref[0, 0]
        neg_inf = jnp.float32(-jnp.inf)
        masked = jnp.where(flag & (conf > _CONF_THRESH), conf, neg_inf)

        iota = (jax.lax.broadcasted_iota(jnp.int32, (_R, _C), 0) * _C
                + jax.lax.broadcasted_iota(jnp.int32, (_R, _C), 1))
        slot = jax.lax.broadcasted_iota(jnp.int32, (1, _S), 1)

        def sel_body(t, carry):
            sc, ts, tx1, ty1, tx2, ty2 = carry
            val = jnp.max(sc)
            idx = jnp.max(jnp.where(sc == val, iota, -1))
            m = iota == idx
            gx1 = jnp.sum(jnp.where(m, bx1, 0.0))
            gy1 = jnp.sum(jnp.where(m, by1, 0.0))
            gx2 = jnp.sum(jnp.where(m, bx2, 0.0))
            gy2 = jnp.sum(jnp.where(m, by2, 0.0))
            sel = slot == t
            ts = jnp.where(sel, val, ts)
            tx1 = jnp.where(sel, gx1, tx1)
            ty1 = jnp.where(sel, gy1, ty1)
            tx2 = jnp.where(sel, gx2, tx2)
            ty2 = jnp.where(sel, gy2, ty2)
            sc = jnp.where(m, neg_inf, sc)
            return sc, ts, tx1, ty1, tx2, ty2

        init = (masked,
                jnp.full((1, _S), neg_inf, jnp.float32),
                jnp.zeros((1, _S), jnp.float32),
                jnp.zeros((1, _S), jnp.float32),
                jnp.zeros((1, _S), jnp.float32),
                jnp.zeros((1, _S), jnp.float32))
        _, ts, tx1, ty1, tx2, ty2 = jax.lax.fori_loop(0, _TOP_K, sel_body, init)

        areav = (tx2 - tx1) * (ty2 - ty1)
        zcol = jnp.zeros((1, _S), jnp.float32)

        def nms_body(t, carry):
            supp, count, os, ox1, oy1, ox2, oy2 = carry
            selt = slot == t
            st = jnp.sum(jnp.where(selt, ts, 0.0))
            x1t = jnp.sum(jnp.where(selt, tx1, 0.0))
            y1t = jnp.sum(jnp.where(selt, ty1, 0.0))
            x2t = jnp.sum(jnp.where(selt, tx2, 0.0))
            y2t = jnp.sum(jnp.where(selt, ty2, 0.0))
            suppt = jnp.sum(jnp.where(selt, supp, 0.0)) > 0.0
            active = (st > _CONF_THRESH) & jnp.logical_not(suppt)

            xx1 = jnp.maximum(x1t, tx1)
            yy1 = jnp.maximum(y1t, ty1)
            xx2 = jnp.minimum(x2t, tx2)
            yy2 = jnp.minimum(y2t, ty2)
            w = jnp.clip(xx2 - xx1, 0.0, None)
            h = jnp.clip(yy2 - yy1, 0.0, None)
            inter = w * h
            areat = (x2t - x1t) * (y2t - y1t)
            iou = inter / (areat + areav - inter + 1e-12)
            supp = jnp.where(active & (iou > _NMS_THRESH) & (slot > t),
                             1.0, supp)

            put = active & (slot == count)
            os = jnp.where(put, st, os)
            ox1 = jnp.where(put, x1t, ox1)
            oy1 = jnp.where(put, y1t, oy1)
            ox2 = jnp.where(put, x2t, ox2)
            oy2 = jnp.where(put, y2t, oy2)
            count = count + active.astype(jnp.int32)
            return supp, count, os, ox1, oy1, ox2, oy2

        supp0 = jnp.zeros((1, _S), jnp.float32)
        _, _, os, ox1, oy1, ox2, oy2 = jax.lax.fori_loop(
            0, _TOP_K, nms_body,
            (supp0, jnp.int32(0), zcol, zcol, zcol, zcol, zcol))

        slab = jnp.concatenate([os, ox1, oy1, ox2, oy2, zcol, zcol, zcol],
                               axis=0)
        out_ref[...] = slab.reshape(1, 1, 8, _S)


def _planes(x, npad):
    b, n, k = x.shape
    xp = jnp.pad(x, ((0, 0), (0, npad - n), (0, 0)))
    return xp.transpose(0, 2, 1).reshape(b, k, _R, _C)


@jax.jit
def kernel(bi_loc_data, bi_conf_data, multi_loc_data, multi_conf_data, prior_data):
    num = bi_loc_data.shape[0]
    bl = _planes(bi_loc_data, _NP)
    ml = _planes(multi_loc_data, _NP)
    bc = _planes(bi_conf_data.reshape(num, _N, 2), _NP)
    cf = _planes(multi_conf_data.reshape(num, _N, _NUM_CLASSES), _NP)
    pr = jnp.pad(prior_data, ((0, _NP - _N), (0, 0))).T.reshape(4, _R, _C)

    slab = pl.pallas_call(
        _detect_body,
        grid=(num, _NUM_CLASSES),
        in_specs=[
            pl.BlockSpec((1, 4, _R, _C), lambda i, c: (i, 0, 0, 0)),
            pl.BlockSpec((1, 2, _R, _C), lambda i, c: (i, 0, 0, 0)),
            pl.BlockSpec((1, 4, _R, _C), lambda i, c: (i, 0, 0, 0)),
            pl.BlockSpec((1, 1, _R, _C), lambda i, c: (i, c, 0, 0)),
            pl.BlockSpec((4, _R, _C), lambda i, c: (0, 0, 0)),
        ],
        out_specs=pl.BlockSpec((1, 1, 8, _S), lambda i, c: (i, c, 0, 0)),
        out_shape=jax.ShapeDtypeStruct((num, _NUM_CLASSES, 8, _S),
                                       jnp.float32),
        compiler_params=pltpu.CompilerParams(
            dimension_semantics=("parallel", "parallel")),
    )(bl, bc, ml, cf, pr)
    return slab[:, :, :5, :_TOP_K].transpose(0, 1, 3, 2)
